# baseline (speedup 1.0000x reference)
import functools

import jax
import jax.numpy as jnp
from jax import lax
from jax.experimental import pallas as pl
from jax.experimental.pallas import tpu as pltpu
from jax.experimental.pallas import tpu_sc as plsc

N = 10000
E = 320000
L = 320000
H = 128
EA = 16

NC = 2
NS = 16
NW = NC * NS
N_PAD = 10112
RPT = N_PAD // NS
CH = 128
NBUF = 5
E_PAD = 327680
DCH = 80
SCH = 160
HH = H // 2
LPW = L // NW
DEGW = 16

_NEG_SLOPE = 0.01


def _leaky(x):
    return jnp.where(x >= 0, x, x * _NEG_SLOPE)


_ROWS = 1000
_GRID = N // _ROWS


def _mlp_body(x_ref, w1t_ref, b1_ref, w2t_ref, b2_ref, o_ref):
    h = _leaky(jnp.dot(x_ref[...], w1t_ref[...],
                       preferred_element_type=jnp.float32) + b1_ref[...])
    o_ref[...] = _leaky(jnp.dot(h, w2t_ref[...],
                                preferred_element_type=jnp.float32) + b2_ref[...])


def _mlp(x, w1t, b1, w2t, b2):
    return pl.pallas_call(
        _mlp_body,
        grid=(_GRID,),
        in_specs=[
            pl.BlockSpec((_ROWS, H), lambda i: (i, 0)),
            pl.BlockSpec((H, H), lambda i: (0, 0)),
            pl.BlockSpec((1, H), lambda i: (0, 0)),
            pl.BlockSpec((H, H), lambda i: (0, 0)),
            pl.BlockSpec((1, H), lambda i: (0, 0)),
        ],
        out_specs=pl.BlockSpec((_ROWS, H), lambda i: (i, 0)),
        out_shape=jax.ShapeDtypeStruct((N, H), jnp.float32),
    )(x, w1t, b1, w2t, b2)


def _dis_from_deg(deg_ref):
    deg = deg_ref[0, :, 0:1] + deg_ref[1, :, 0:1] + 1.0
    return lax.rsqrt(deg)


def _gmsg_body(h_ref, cw_ref, deg_ref, g_ref):
    dis = _dis_from_deg(deg_ref)
    gm = jnp.dot(h_ref[...], cw_ref[...],
                 preferred_element_type=jnp.float32) * dis
    g_ref[0] = gm[:, :HH]
    g_ref[1] = gm[:, HH:]


def _gmsg(h, cw, deg2):
    return pl.pallas_call(
        _gmsg_body,
        grid=(_GRID,),
        in_specs=[
            pl.BlockSpec((_ROWS, H), lambda i: (i, 0)),
            pl.BlockSpec((H, H), lambda i: (0, 0)),
            pl.BlockSpec((2, _ROWS, DEGW), lambda i: (0, i, 0)),
        ],
        out_specs=pl.BlockSpec((2, _ROWS, HH), lambda i: (0, i, 0)),
        out_shape=jax.ShapeDtypeStruct((2, N, HH), jnp.float32),
    )(h, cw, deg2)


def _gru_body(acc_ref, g_ref, deg_ref, cb_ref, prev_ref, wiht_ref, whht_ref,
              bih_ref, bhh_ref, o_ref):
    dis = _dis_from_deg(deg_ref)
    full = jnp.concatenate([acc_ref[0] + g_ref[0], acc_ref[1] + g_ref[1]],
                           axis=1)
    conv = dis * full + cb_ref[...]
    a = _leaky(conv)
    prev = prev_ref[...]
    gi = jnp.dot(a, wiht_ref[...], preferred_element_type=jnp.float32) + bih_ref[...]
    gh = jnp.dot(prev, whht_ref[...], preferred_element_type=jnp.float32) + bhh_ref[...]
    i_r, i_z, i_n = gi[:, :H], gi[:, H:2 * H], gi[:, 2 * H:]
    h_r, h_z, h_n = gh[:, :H], gh[:, H:2 * H], gh[:, 2 * H:]
    r = jax.nn.sigmoid(i_r + h_r)
    z = jax.nn.sigmoid(i_z + h_z)
    n = jnp.tanh(i_n + r * h_n)
    o_ref[...] = (1.0 - z) * n + z * prev


def _gru(acc2, g, deg2, cb, prev, wiht, whht, bih, bhh):
    return pl.pallas_call(
        _gru_body,
        grid=(_GRID,),
        in_specs=[
            pl.BlockSpec((2, _ROWS, HH), lambda i: (0, i, 0)),
            pl.BlockSpec((2, _ROWS, HH), lambda i: (0, i, 0)),
            pl.BlockSpec((2, _ROWS, DEGW), lambda i: (0, i, 0)),
            pl.BlockSpec((1, H), lambda i: (0, 0)),
            pl.BlockSpec((_ROWS, H), lambda i: (i, 0)),
            pl.BlockSpec((H, 3 * H), lambda i: (0, 0)),
            pl.BlockSpec((H, 3 * H), lambda i: (0, 0)),
            pl.BlockSpec((1, 3 * H), lambda i: (0, 0)),
            pl.BlockSpec((1, 3 * H), lambda i: (0, 0)),
        ],
        out_specs=pl.BlockSpec((_ROWS, H), lambda i: (i, 0)),
        out_shape=jax.ShapeDtypeStruct((N, H), jnp.float32),
    )(acc2, g, deg2, cb, prev, wiht, whht, bih, bhh)


def _proj_body(h_ref, w_ref, o_ref):
    o_ref[...] = jnp.dot(h_ref[...], w_ref[...],
                         preferred_element_type=jnp.float32)


def _proj(h, wsd):
    return pl.pallas_call(
        _proj_body,
        grid=(_GRID,),
        in_specs=[
            pl.BlockSpec((_ROWS, H), lambda i: (i, 0)),
            pl.BlockSpec((H, 8), lambda i: (0, 0)),
        ],
        out_specs=pl.BlockSpec((_ROWS, 8), lambda i: (i, 0)),
        out_shape=jax.ShapeDtypeStruct((N, 8), jnp.float32),
    )(h, wsd)


_EROWS = 8000


def _eap_body(ea_ref, w_ref, bp_ref, o_ref):
    o_ref[...] = jnp.dot(ea_ref[...], w_ref[...],
                         preferred_element_type=jnp.float32) + bp_ref[...]


def _eap(edge_attr, wpe, bp):
    return pl.pallas_call(
        _eap_body,
        grid=(L // _EROWS,),
        in_specs=[
            pl.BlockSpec((_EROWS, EA), lambda i: (i, 0)),
            pl.BlockSpec((EA, 8), lambda i: (0, 0)),
            pl.BlockSpec((1, 8), lambda i: (0, 0)),
        ],
        out_specs=pl.BlockSpec((_EROWS, 8), lambda i: (i, 0)),
        out_shape=jax.ShapeDtypeStruct((L, 8), jnp.float32),
    )(edge_attr, wpe, bp)


_MESH = functools.partial(plsc.VectorSubcoreMesh,
                          core_axis_name="c", subcore_axis_name="s",
                          num_cores=NC, num_subcores=NS)


def _zero_rows(buf, nrows, width):
    def body(i, _):
        for k in range(width // 16):
            buf[i, pl.ds(k * 16, 16)] = jnp.zeros((16,), jnp.float32)
        return 0
    lax.fori_loop(0, nrows, body, 0, unroll=2)


def _fill_tile_rows(accum, src2d, base):
    for off in (0, 128, 256, 384):
        pltpu.sync_copy(src2d, accum.at[pl.ds(base + off, 128)])
    pltpu.sync_copy(src2d.at[pl.ds(0, 120)], accum.at[pl.ds(base + 512, 120)])


def _deg_body(dst_hbm, out_hbm, dst_v, ones_v, zbuf, accum, sem):
    cid = lax.axis_index("c")
    sid = lax.axis_index("s")
    wid = cid * NS + sid
    base = sid * RPT

    _zero_rows(zbuf, CH, DEGW)
    def ones_body(i, _):
        ones_v[i, pl.ds(0, DEGW)] = jnp.ones((DEGW,), jnp.float32)
        return 0
    lax.fori_loop(0, CH, ones_body, 0, unroll=2)
    _fill_tile_rows(accum, zbuf, base)
    pltpu.sync_copy(dst_hbm.at[wid], dst_v)
    plsc.subcore_barrier()

    def body(j, _):
        pltpu.sync_copy(ones_v, accum.at[dst_v.at[j]], add=True)
        return 0
    lax.fori_loop(0, DCH, body, 0)
    plsc.subcore_barrier()
    pltpu.sync_copy(accum.at[pl.ds(base, RPT)],
                    out_hbm.at[cid, pl.ds(base, RPT)])


def _deg(dst_rs):
    k = pl.kernel(
        _deg_body,
        out_type=jax.ShapeDtypeStruct((NC, N_PAD, DEGW), jnp.float32),
        mesh=_MESH(),
        compiler_params=pltpu.CompilerParams(use_tc_tiling_on_sc=False),
        scratch_types=[
            pltpu.VMEM((DCH, CH), jnp.int32),
            pltpu.VMEM((CH, DEGW), jnp.float32),
            pltpu.VMEM((CH, DEGW), jnp.float32),
            pltpu.VMEM_SHARED((N_PAD, DEGW), jnp.float32),
            pltpu.SemaphoreType.DMA,
        ],
    )
    return k(dst_rs)


def _scat_body(g_hbm, src_hbm, dst_hbm, out_hbm, src_v, dst_v, rowbuf, accum,
               *sems):
    cid = lax.axis_index("c")
    sid = lax.axis_index("s")
    base = sid * RPT
    gsems, ssems = sems[:NBUF], sems[NBUF:]

    _zero_rows(rowbuf.at[0], CH, HH)
    _fill_tile_rows(accum, rowbuf.at[0], base)
    pltpu.sync_copy(src_hbm.at[sid], src_v)
    pltpu.sync_copy(dst_hbm.at[sid], dst_v)
    off = cid * N

    def adj(j, _):
        for k in range(CH // 16):
            s = src_v[j, pl.ds(k * 16, 16)]
            src_v[j, pl.ds(k * 16, 16)] = s + off
        return 0
    lax.fori_loop(0, SCH, adj, 0, unroll=2)
    plsc.subcore_barrier()

    def body(jo, _):
        j = jo * NBUF
        gs, ss = [], []
        for b in range(NBUF):
            gs.append(pltpu.async_copy(g_hbm.at[src_v.at[j + b]],
                                       rowbuf.at[b], gsems[b]))
        for b in range(NBUF):
            gs[b].wait()
            ss.append(pltpu.async_copy(rowbuf.at[b],
                                       accum.at[dst_v.at[j + b]],
                                       ssems[b], add=True))
        for b in range(NBUF):
            ss[b].wait()
        return 0
    lax.fori_loop(0, SCH // NBUF, body, 0)
    plsc.subcore_barrier()
    pltpu.sync_copy(accum.at[pl.ds(base, RPT)],
                    out_hbm.at[cid, pl.ds(base, RPT)])


def _scat(g2, src_rs, dst_rs):
    k = pl.kernel(
        _scat_body,
        out_type=jax.ShapeDtypeStruct((NC, N_PAD, HH), jnp.float32),
        mesh=_MESH(),
        compiler_params=pltpu.CompilerParams(use_tc_tiling_on_sc=False),
        scratch_types=[
            pltpu.VMEM((SCH, CH), jnp.int32),
            pltpu.VMEM((SCH, CH), jnp.int32),
            pltpu.VMEM((NBUF, CH, HH), jnp.float32),
            pltpu.VMEM_SHARED((N_PAD, HH), jnp.float32),
        ] + [pltpu.SemaphoreType.DMA] * (2 * NBUF),
    )
    return k(g2.reshape(2 * N, HH), src_rs, dst_rs)


def _final_body(ps_hbm, pd_hbm, sl_hbm, dl_hbm, ea_hbm, out_hbm,
                ps_v, pd_v, sl_v, dl_v, ea_v, out_v, sem):
    cid = lax.axis_index("c")
    sid = lax.axis_index("s")
    wid = cid * NS + sid

    pltpu.sync_copy(ps_hbm, ps_v)
    pltpu.sync_copy(pd_hbm, pd_v)
    pltpu.sync_copy(sl_hbm.at[wid], sl_v)
    pltpu.sync_copy(dl_hbm.at[wid], dl_v)
    pltpu.sync_copy(ea_hbm.at[wid], ea_v)

    def body(j, _):
        i16 = j * 16
        s_idx = sl_v[pl.ds(i16, 16)]
        d_idx = dl_v[pl.ds(i16, 16)]
        a = plsc.load_gather(ps_v, [s_idx])
        b = plsc.load_gather(pd_v, [d_idx])
        out_v[pl.ds(i16, 16)] = a + b + ea_v[pl.ds(i16, 16)]
        return 0
    lax.fori_loop(0, LPW // 16, body, 0, unroll=4)
    pltpu.sync_copy(out_v, out_hbm.at[wid])


def _final(ps, pd, sl_rs, dl_rs, ea_rs):
    k = pl.kernel(
        _final_body,
        out_type=jax.ShapeDtypeStruct((NW, LPW), jnp.float32),
        mesh=_MESH(),
        compiler_params=pltpu.CompilerParams(needs_layout_passes=False),
        scratch_types=[
            pltpu.VMEM((N,), jnp.float32),
            pltpu.VMEM((N,), jnp.float32),
            pltpu.VMEM((LPW,), jnp.int32),
            pltpu.VMEM((LPW,), jnp.int32),
            pltpu.VMEM((LPW,), jnp.float32),
            pltpu.VMEM((LPW,), jnp.float32),
            pltpu.SemaphoreType.DMA,
        ],
    )
    return k(ps, pd, sl_rs, dl_rs, ea_rs)


def kernel(x, edge_index, edge_label_index, edge_attr, W1, b1, W2, b2,
           convW0, convb0, Wih0, Whh0, bih0, bhh0, prev0,
           convW1, convb1, Wih1, Whh1, bih1, bhh1, prev1, Wp, bp):
    f32 = jnp.float32

    src = edge_index[0]
    dst = edge_index[1]
    pad = E_PAD - E
    src_p = jnp.concatenate([src, jnp.zeros((pad,), jnp.int32)])
    dst_p = jnp.concatenate([dst, jnp.full((pad,), N, jnp.int32)])
    src_rs = src_p.reshape(NS, SCH, CH)
    dst_rs = dst_p.reshape(NS, SCH, CH)
    dst_rs32 = dst_p.reshape(NW, DCH, CH)
    sl_rs = edge_label_index[0].reshape(NW, LPW)
    dl_rs = edge_label_index[1].reshape(NW, LPW)

    b1r = b1.reshape(1, H)
    b2r = b2.reshape(1, H)
    wsd = jnp.concatenate(
        [Wp[0, :H].reshape(H, 1), Wp[0, H:2 * H].reshape(H, 1),
         jnp.zeros((H, 6), f32)], axis=1)
    wpe = jnp.concatenate([Wp[0, 2 * H:].reshape(EA, 1),
                           jnp.zeros((EA, 7), f32)], axis=1)
    bp8 = jnp.concatenate([bp.reshape(1, 1), jnp.zeros((1, 7), f32)], axis=1)

    deg2 = _deg(dst_rs32)
    h = _mlp(x, W1.T, b1r, W2.T, b2r)

    embs = []
    for cw, cb, wih, whh, bih, bhh, prev in (
            (convW0, convb0, Wih0, Whh0, bih0, bhh0, prev0),
            (convW1, convb1, Wih1, Whh1, bih1, bhh1, prev1)):
        g = _gmsg(h, cw, deg2)
        acc2 = _scat(g, src_rs, dst_rs)
        h = _gru(acc2, g, deg2, cb.reshape(1, H), prev,
                 wih.T, whh.T, bih.reshape(1, 3 * H), bhh.reshape(1, 3 * H))
        embs.append(h)

    p2 = _proj(h, wsd)
    eap = _eap(edge_attr, wpe, bp8)
    out = _final(p2[:, 0], p2[:, 1], sl_rs, dl_rs,
                 eap[:, 0].reshape(NW, LPW)).reshape(L)
    return (out, embs[0], embs[1])


# speedup vs baseline: 1.4486x; 1.4486x over previous
import functools

import jax
import jax.numpy as jnp
from jax import lax
from jax.experimental import pallas as pl
from jax.experimental.pallas import tpu as pltpu
from jax.experimental.pallas import tpu_sc as plsc

N = 10000
E = 320000
L = 320000
H = 128
EA = 16

NC = 2
NS = 16
NW = NC * NS
N_PAD = 10112
RPT = N_PAD // NS
CH = 128
NBUF = 5
E_PAD = 327680
DCH = 80
SCH = 160
HH = H // 2
QW = H // 4
LPW = L // NW
DEGW = 16

_NEG_SLOPE = 0.01


def _leaky(x):
    return jnp.where(x >= 0, x, x * _NEG_SLOPE)


_ROWS = 1000
_GRID = N // _ROWS


def _mlp_body(x_ref, w1t_ref, b1_ref, w2t_ref, b2_ref, o_ref):
    h = _leaky(jnp.dot(x_ref[...], w1t_ref[...],
                       preferred_element_type=jnp.float32) + b1_ref[...])
    o_ref[...] = _leaky(jnp.dot(h, w2t_ref[...],
                                preferred_element_type=jnp.float32) + b2_ref[...])


def _mlp(x, w1t, b1, w2t, b2):
    return pl.pallas_call(
        _mlp_body,
        grid=(_GRID,),
        in_specs=[
            pl.BlockSpec((_ROWS, H), lambda i: (i, 0)),
            pl.BlockSpec((H, H), lambda i: (0, 0)),
            pl.BlockSpec((1, H), lambda i: (0, 0)),
            pl.BlockSpec((H, H), lambda i: (0, 0)),
            pl.BlockSpec((1, H), lambda i: (0, 0)),
        ],
        out_specs=pl.BlockSpec((_ROWS, H), lambda i: (i, 0)),
        out_shape=jax.ShapeDtypeStruct((N, H), jnp.float32),
    )(x, w1t, b1, w2t, b2)


def _dis_from_deg(deg_ref):
    deg = deg_ref[0, :, 0:1] + deg_ref[1, :, 0:1] + 1.0
    return lax.rsqrt(deg)


def _gmsg_body(h_ref, cw_ref, deg_ref, g_ref):
    dis = _dis_from_deg(deg_ref)
    gm = jnp.dot(h_ref[...], cw_ref[...],
                 preferred_element_type=jnp.float32) * dis
    for q in range(4):
        g_ref[q] = gm[:, q * QW:(q + 1) * QW]


def _gmsg(h, cw, deg2):
    return pl.pallas_call(
        _gmsg_body,
        grid=(_GRID,),
        in_specs=[
            pl.BlockSpec((_ROWS, H), lambda i: (i, 0)),
            pl.BlockSpec((H, H), lambda i: (0, 0)),
            pl.BlockSpec((2, _ROWS, DEGW), lambda i: (0, i, 0)),
        ],
        out_specs=pl.BlockSpec((4, _ROWS, QW), lambda i: (0, i, 0)),
        out_shape=jax.ShapeDtypeStruct((4, N, QW), jnp.float32),
    )(h, cw, deg2)


def _gru_body(acc_ref, g_ref, deg_ref, cb_ref, prev_ref, wiht_ref, whht_ref,
              bih_ref, bhh_ref, o_ref):
    dis = _dis_from_deg(deg_ref)
    full = jnp.concatenate([acc_ref[q] + g_ref[q] for q in range(4)],
                           axis=1)
    conv = dis * full + cb_ref[...]
    a = _leaky(conv)
    prev = prev_ref[...]
    gi = jnp.dot(a, wiht_ref[...], preferred_element_type=jnp.float32) + bih_ref[...]
    gh = jnp.dot(prev, whht_ref[...], preferred_element_type=jnp.float32) + bhh_ref[...]
    i_r, i_z, i_n = gi[:, :H], gi[:, H:2 * H], gi[:, 2 * H:]
    h_r, h_z, h_n = gh[:, :H], gh[:, H:2 * H], gh[:, 2 * H:]
    r = jax.nn.sigmoid(i_r + h_r)
    z = jax.nn.sigmoid(i_z + h_z)
    n = jnp.tanh(i_n + r * h_n)
    o_ref[...] = (1.0 - z) * n + z * prev


def _gru(acc2, g, deg2, cb, prev, wiht, whht, bih, bhh):
    return pl.pallas_call(
        _gru_body,
        grid=(_GRID,),
        in_specs=[
            pl.BlockSpec((4, _ROWS, QW), lambda i: (0, i, 0)),
            pl.BlockSpec((4, _ROWS, QW), lambda i: (0, i, 0)),
            pl.BlockSpec((2, _ROWS, DEGW), lambda i: (0, i, 0)),
            pl.BlockSpec((1, H), lambda i: (0, 0)),
            pl.BlockSpec((_ROWS, H), lambda i: (i, 0)),
            pl.BlockSpec((H, 3 * H), lambda i: (0, 0)),
            pl.BlockSpec((H, 3 * H), lambda i: (0, 0)),
            pl.BlockSpec((1, 3 * H), lambda i: (0, 0)),
            pl.BlockSpec((1, 3 * H), lambda i: (0, 0)),
        ],
        out_specs=pl.BlockSpec((_ROWS, H), lambda i: (i, 0)),
        out_shape=jax.ShapeDtypeStruct((N, H), jnp.float32),
    )(acc2, g, deg2, cb, prev, wiht, whht, bih, bhh)


def _proj_body(h_ref, w_ref, o_ref):
    o_ref[...] = jnp.dot(h_ref[...], w_ref[...],
                         preferred_element_type=jnp.float32)


def _proj(h, wsd):
    return pl.pallas_call(
        _proj_body,
        grid=(_GRID,),
        in_specs=[
            pl.BlockSpec((_ROWS, H), lambda i: (i, 0)),
            pl.BlockSpec((H, 8), lambda i: (0, 0)),
        ],
        out_specs=pl.BlockSpec((_ROWS, 8), lambda i: (i, 0)),
        out_shape=jax.ShapeDtypeStruct((N, 8), jnp.float32),
    )(h, wsd)


_EROWS = 8000


def _eap_body(ea_ref, w_ref, bp_ref, o_ref):
    o_ref[...] = jnp.dot(ea_ref[...], w_ref[...],
                         preferred_element_type=jnp.float32) + bp_ref[...]


def _eap(edge_attr, wpe, bp):
    return pl.pallas_call(
        _eap_body,
        grid=(L // _EROWS,),
        in_specs=[
            pl.BlockSpec((_EROWS, EA), lambda i: (i, 0)),
            pl.BlockSpec((EA, 8), lambda i: (0, 0)),
            pl.BlockSpec((1, 8), lambda i: (0, 0)),
        ],
        out_specs=pl.BlockSpec((_EROWS, 8), lambda i: (i, 0)),
        out_shape=jax.ShapeDtypeStruct((L, 8), jnp.float32),
    )(edge_attr, wpe, bp)


_MESH = functools.partial(plsc.VectorSubcoreMesh,
                          core_axis_name="c", subcore_axis_name="s",
                          num_cores=NC, num_subcores=NS)


def _zero_rows(buf, nrows, width):
    def body(i, _):
        for k in range(width // 16):
            buf[i, pl.ds(k * 16, 16)] = jnp.zeros((16,), jnp.float32)
        return 0
    lax.fori_loop(0, nrows, body, 0, unroll=2)


def _fill_tile_rows(accum, src2d, base):
    for off in (0, 128, 256, 384):
        pltpu.sync_copy(src2d, accum.at[pl.ds(base + off, 128)])
    pltpu.sync_copy(src2d.at[pl.ds(0, 120)], accum.at[pl.ds(base + 512, 120)])


def _deg_body(dst_hbm, out_hbm, dst_v, ones_v, zbuf, accum, sem):
    cid = lax.axis_index("c")
    sid = lax.axis_index("s")
    wid = cid * NS + sid
    base = sid * RPT

    _zero_rows(zbuf, CH, DEGW)
    def ones_body(i, _):
        ones_v[i, pl.ds(0, DEGW)] = jnp.ones((DEGW,), jnp.float32)
        return 0
    lax.fori_loop(0, CH, ones_body, 0, unroll=2)
    _fill_tile_rows(accum, zbuf, base)
    pltpu.sync_copy(dst_hbm.at[wid], dst_v)
    plsc.subcore_barrier()

    def body(j, _):
        pltpu.sync_copy(ones_v, accum.at[dst_v.at[j]], add=True)
        return 0
    lax.fori_loop(0, DCH, body, 0)
    plsc.subcore_barrier()
    pltpu.sync_copy(accum.at[pl.ds(base, RPT)],
                    out_hbm.at[cid, pl.ds(base, RPT)])


def _deg(dst_rs):
    k = pl.kernel(
        _deg_body,
        out_type=jax.ShapeDtypeStruct((NC, N_PAD, DEGW), jnp.float32),
        mesh=_MESH(),
        compiler_params=pltpu.CompilerParams(use_tc_tiling_on_sc=False),
        scratch_types=[
            pltpu.VMEM((DCH, CH), jnp.int32),
            pltpu.VMEM((CH, DEGW), jnp.float32),
            pltpu.VMEM((CH, DEGW), jnp.float32),
            pltpu.VMEM_SHARED((N_PAD, DEGW), jnp.float32),
            pltpu.SemaphoreType.DMA,
        ],
    )
    return k(dst_rs)


def _scat_body(g_hbm, src_hbm, dst_hbm, out_hbm, src_v, dst_v, rowbuf, zbuf,
               accum, gstage, *sems):
    cid = lax.axis_index("c")
    sid = lax.axis_index("s")
    base = sid * RPT
    gsems, ssems = sems[:NBUF], sems[NBUF:]
    rpt_n = N // NS

    _zero_rows(zbuf, CH, QW)
    pltpu.sync_copy(src_hbm.at[sid], src_v)
    pltpu.sync_copy(dst_hbm.at[sid], dst_v)

    for qi in range(2):
        q = cid * 2 + qi
        pltpu.sync_copy(g_hbm.at[q, pl.ds(sid * rpt_n, rpt_n)],
                        gstage.at[pl.ds(sid * rpt_n, rpt_n)])
        _fill_tile_rows(accum, zbuf, base)
        plsc.subcore_barrier()

        def body(jo, _):
            j = jo * NBUF
            gs, ss = [], []
            for b in range(NBUF):
                gs.append(pltpu.async_copy(gstage.at[src_v.at[j + b]],
                                           rowbuf.at[b], gsems[b]))
            for b in range(NBUF):
                gs[b].wait()
                ss.append(pltpu.async_copy(rowbuf.at[b],
                                           accum.at[dst_v.at[j + b]],
                                           ssems[b], add=True))
            for b in range(NBUF):
                ss[b].wait()
            return 0
        lax.fori_loop(0, SCH // NBUF, body, 0)
        plsc.subcore_barrier()
        pltpu.sync_copy(accum.at[pl.ds(base, RPT)],
                        out_hbm.at[q, pl.ds(base, RPT)])
        plsc.subcore_barrier()


def _scat(g4, src_rs, dst_rs):
    k = pl.kernel(
        _scat_body,
        out_type=jax.ShapeDtypeStruct((4, N_PAD, QW), jnp.float32),
        mesh=_MESH(),
        compiler_params=pltpu.CompilerParams(use_tc_tiling_on_sc=False),
        scratch_types=[
            pltpu.VMEM((SCH, CH), jnp.int32),
            pltpu.VMEM((SCH, CH), jnp.int32),
            pltpu.VMEM((NBUF, CH, QW), jnp.float32),
            pltpu.VMEM((CH, QW), jnp.float32),
            pltpu.VMEM_SHARED((N_PAD, QW), jnp.float32),
            pltpu.VMEM_SHARED((N, QW), jnp.float32),
        ] + [pltpu.SemaphoreType.DMA] * (2 * NBUF),
    )
    return k(g4, src_rs, dst_rs)


def _final_body(ps_hbm, pd_hbm, sl_hbm, dl_hbm, ea_hbm, out_hbm,
                ps_v, pd_v, sl_v, dl_v, ea_v, out_v, sem):
    cid = lax.axis_index("c")
    sid = lax.axis_index("s")
    wid = cid * NS + sid

    pltpu.sync_copy(ps_hbm, ps_v)
    pltpu.sync_copy(pd_hbm, pd_v)
    pltpu.sync_copy(sl_hbm.at[wid], sl_v)
    pltpu.sync_copy(dl_hbm.at[wid], dl_v)
    pltpu.sync_copy(ea_hbm.at[wid], ea_v)

    def body(j, _):
        i16 = j * 16
        s_idx = sl_v[pl.ds(i16, 16)]
        d_idx = dl_v[pl.ds(i16, 16)]
        a = plsc.load_gather(ps_v, [s_idx])
        b = plsc.load_gather(pd_v, [d_idx])
        out_v[pl.ds(i16, 16)] = a + b + ea_v[pl.ds(i16, 16)]
        return 0
    lax.fori_loop(0, LPW // 16, body, 0, unroll=4)
    pltpu.sync_copy(out_v, out_hbm.at[wid])


def _final(ps, pd, sl_rs, dl_rs, ea_rs):
    k = pl.kernel(
        _final_body,
        out_type=jax.ShapeDtypeStruct((NW, LPW), jnp.float32),
        mesh=_MESH(),
        compiler_params=pltpu.CompilerParams(needs_layout_passes=False),
        scratch_types=[
            pltpu.VMEM((N,), jnp.float32),
            pltpu.VMEM((N,), jnp.float32),
            pltpu.VMEM((LPW,), jnp.int32),
            pltpu.VMEM((LPW,), jnp.int32),
            pltpu.VMEM((LPW,), jnp.float32),
            pltpu.VMEM((LPW,), jnp.float32),
            pltpu.SemaphoreType.DMA,
        ],
    )
    return k(ps, pd, sl_rs, dl_rs, ea_rs)


def kernel(x, edge_index, edge_label_index, edge_attr, W1, b1, W2, b2,
           convW0, convb0, Wih0, Whh0, bih0, bhh0, prev0,
           convW1, convb1, Wih1, Whh1, bih1, bhh1, prev1, Wp, bp):
    f32 = jnp.float32

    src = edge_index[0]
    dst = edge_index[1]
    pad = E_PAD - E
    src_p = jnp.concatenate([src, jnp.zeros((pad,), jnp.int32)])
    dst_p = jnp.concatenate([dst, jnp.full((pad,), N, jnp.int32)])
    src_rs = src_p.reshape(NS, SCH, CH)
    dst_rs = dst_p.reshape(NS, SCH, CH)
    dst_rs32 = dst_p.reshape(NW, DCH, CH)
    sl_rs = edge_label_index[0].reshape(NW, LPW)
    dl_rs = edge_label_index[1].reshape(NW, LPW)

    b1r = b1.reshape(1, H)
    b2r = b2.reshape(1, H)
    wsd = jnp.concatenate(
        [Wp[0, :H].reshape(H, 1), Wp[0, H:2 * H].reshape(H, 1),
         jnp.zeros((H, 6), f32)], axis=1)
    wpe = jnp.concatenate([Wp[0, 2 * H:].reshape(EA, 1),
                           jnp.zeros((EA, 7), f32)], axis=1)
    bp8 = jnp.concatenate([bp.reshape(1, 1), jnp.zeros((1, 7), f32)], axis=1)

    deg2 = _deg(dst_rs32)
    h = _mlp(x, W1.T, b1r, W2.T, b2r)

    embs = []
    for cw, cb, wih, whh, bih, bhh, prev in (
            (convW0, convb0, Wih0, Whh0, bih0, bhh0, prev0),
            (convW1, convb1, Wih1, Whh1, bih1, bhh1, prev1)):
        g = _gmsg(h, cw, deg2)
        acc2 = _scat(g, src_rs, dst_rs)
        h = _gru(acc2, g, deg2, cb.reshape(1, H), prev,
                 wih.T, whh.T, bih.reshape(1, 3 * H), bhh.reshape(1, 3 * H))
        embs.append(h)

    p2 = _proj(h, wsd)
    eap = _eap(edge_attr, wpe, bp8)
    out = _final(p2[:, 0], p2[:, 1], sl_rs, dl_rs,
                 eap[:, 0].reshape(NW, LPW)).reshape(L)
    return (out, embs[0], embs[1])


# speedup vs baseline: 1.4586x; 1.0069x over previous
import functools

import jax
import jax.numpy as jnp
from jax import lax
from jax.experimental import pallas as pl
from jax.experimental.pallas import tpu as pltpu
from jax.experimental.pallas import tpu_sc as plsc

N = 10000
E = 320000
L = 320000
H = 128
EA = 16

NC = 2
NS = 16
NW = NC * NS
N_PAD = 10112
RPT = N_PAD // NS
CH = 128
NBUF = 8
E_PAD = 327680
DCH = 80
SCH = 160
HH = H // 2
QW = H // 4
LPW = L // NW
DEGW = 16

_NEG_SLOPE = 0.01


def _leaky(x):
    return jnp.where(x >= 0, x, x * _NEG_SLOPE)


_ROWS = 1000
_GRID = N // _ROWS


def _mlp_body(x_ref, w1t_ref, b1_ref, w2t_ref, b2_ref, o_ref):
    h = _leaky(jnp.dot(x_ref[...], w1t_ref[...],
                       preferred_element_type=jnp.float32) + b1_ref[...])
    o_ref[...] = _leaky(jnp.dot(h, w2t_ref[...],
                                preferred_element_type=jnp.float32) + b2_ref[...])


def _mlp(x, w1t, b1, w2t, b2):
    return pl.pallas_call(
        _mlp_body,
        grid=(_GRID,),
        in_specs=[
            pl.BlockSpec((_ROWS, H), lambda i: (i, 0)),
            pl.BlockSpec((H, H), lambda i: (0, 0)),
            pl.BlockSpec((1, H), lambda i: (0, 0)),
            pl.BlockSpec((H, H), lambda i: (0, 0)),
            pl.BlockSpec((1, H), lambda i: (0, 0)),
        ],
        out_specs=pl.BlockSpec((_ROWS, H), lambda i: (i, 0)),
        out_shape=jax.ShapeDtypeStruct((N, H), jnp.float32),
    )(x, w1t, b1, w2t, b2)


def _dis_from_deg(deg_ref):
    deg = deg_ref[0, :, 0:1] + deg_ref[1, :, 0:1] + 1.0
    return lax.rsqrt(deg)


def _gmsg_body(h_ref, cw_ref, deg_ref, g_ref):
    dis = _dis_from_deg(deg_ref)
    gm = jnp.dot(h_ref[...], cw_ref[...],
                 preferred_element_type=jnp.float32) * dis
    for q in range(4):
        g_ref[q] = gm[:, q * QW:(q + 1) * QW]


def _gmsg(h, cw, deg2):
    return pl.pallas_call(
        _gmsg_body,
        grid=(_GRID,),
        in_specs=[
            pl.BlockSpec((_ROWS, H), lambda i: (i, 0)),
            pl.BlockSpec((H, H), lambda i: (0, 0)),
            pl.BlockSpec((2, _ROWS, DEGW), lambda i: (0, i, 0)),
        ],
        out_specs=pl.BlockSpec((4, _ROWS, QW), lambda i: (0, i, 0)),
        out_shape=jax.ShapeDtypeStruct((4, N, QW), jnp.float32),
    )(h, cw, deg2)


def _gru_body(acc_ref, g_ref, deg_ref, cb_ref, prev_ref, wiht_ref, whht_ref,
              bih_ref, bhh_ref, o_ref):
    dis = _dis_from_deg(deg_ref)
    full = jnp.concatenate([acc_ref[q] + g_ref[q] for q in range(4)],
                           axis=1)
    conv = dis * full + cb_ref[...]
    a = _leaky(conv)
    prev = prev_ref[...]
    gi = jnp.dot(a, wiht_ref[...], preferred_element_type=jnp.float32) + bih_ref[...]
    gh = jnp.dot(prev, whht_ref[...], preferred_element_type=jnp.float32) + bhh_ref[...]
    i_r, i_z, i_n = gi[:, :H], gi[:, H:2 * H], gi[:, 2 * H:]
    h_r, h_z, h_n = gh[:, :H], gh[:, H:2 * H], gh[:, 2 * H:]
    r = jax.nn.sigmoid(i_r + h_r)
    z = jax.nn.sigmoid(i_z + h_z)
    n = jnp.tanh(i_n + r * h_n)
    o_ref[...] = (1.0 - z) * n + z * prev


def _gru(acc2, g, deg2, cb, prev, wiht, whht, bih, bhh):
    return pl.pallas_call(
        _gru_body,
        grid=(_GRID,),
        in_specs=[
            pl.BlockSpec((4, _ROWS, QW), lambda i: (0, i, 0)),
            pl.BlockSpec((4, _ROWS, QW), lambda i: (0, i, 0)),
            pl.BlockSpec((2, _ROWS, DEGW), lambda i: (0, i, 0)),
            pl.BlockSpec((1, H), lambda i: (0, 0)),
            pl.BlockSpec((_ROWS, H), lambda i: (i, 0)),
            pl.BlockSpec((H, 3 * H), lambda i: (0, 0)),
            pl.BlockSpec((H, 3 * H), lambda i: (0, 0)),
            pl.BlockSpec((1, 3 * H), lambda i: (0, 0)),
            pl.BlockSpec((1, 3 * H), lambda i: (0, 0)),
        ],
        out_specs=pl.BlockSpec((_ROWS, H), lambda i: (i, 0)),
        out_shape=jax.ShapeDtypeStruct((N, H), jnp.float32),
    )(acc2, g, deg2, cb, prev, wiht, whht, bih, bhh)


def _proj_body(h_ref, w_ref, o_ref):
    o_ref[...] = jnp.dot(h_ref[...], w_ref[...],
                         preferred_element_type=jnp.float32)


def _proj(h, wsd):
    return pl.pallas_call(
        _proj_body,
        grid=(_GRID,),
        in_specs=[
            pl.BlockSpec((_ROWS, H), lambda i: (i, 0)),
            pl.BlockSpec((H, 8), lambda i: (0, 0)),
        ],
        out_specs=pl.BlockSpec((_ROWS, 8), lambda i: (i, 0)),
        out_shape=jax.ShapeDtypeStruct((N, 8), jnp.float32),
    )(h, wsd)


_EROWS = 8000


def _eap_body(ea_ref, w_ref, bp_ref, o_ref):
    o_ref[...] = jnp.dot(ea_ref[...], w_ref[...],
                         preferred_element_type=jnp.float32) + bp_ref[...]


def _eap(edge_attr, wpe, bp):
    return pl.pallas_call(
        _eap_body,
        grid=(L // _EROWS,),
        in_specs=[
            pl.BlockSpec((_EROWS, EA), lambda i: (i, 0)),
            pl.BlockSpec((EA, 8), lambda i: (0, 0)),
            pl.BlockSpec((1, 8), lambda i: (0, 0)),
        ],
        out_specs=pl.BlockSpec((_EROWS, 8), lambda i: (i, 0)),
        out_shape=jax.ShapeDtypeStruct((L, 8), jnp.float32),
    )(edge_attr, wpe, bp)


_MESH = functools.partial(plsc.VectorSubcoreMesh,
                          core_axis_name="c", subcore_axis_name="s",
                          num_cores=NC, num_subcores=NS)


def _zero_rows(buf, nrows, width):
    def body(i, _):
        for k in range(width // 16):
            buf[i, pl.ds(k * 16, 16)] = jnp.zeros((16,), jnp.float32)
        return 0
    lax.fori_loop(0, nrows, body, 0, unroll=2)


def _fill_tile_rows(accum, src2d, base):
    for off in (0, 128, 256, 384):
        pltpu.sync_copy(src2d, accum.at[pl.ds(base + off, 128)])
    pltpu.sync_copy(src2d.at[pl.ds(0, 120)], accum.at[pl.ds(base + 512, 120)])


def _deg_body(dst_hbm, out_hbm, dst_v, ones_v, zbuf, accum, sem):
    cid = lax.axis_index("c")
    sid = lax.axis_index("s")
    wid = cid * NS + sid
    base = sid * RPT

    _zero_rows(zbuf, CH, DEGW)
    def ones_body(i, _):
        ones_v[i, pl.ds(0, DEGW)] = jnp.ones((DEGW,), jnp.float32)
        return 0
    lax.fori_loop(0, CH, ones_body, 0, unroll=2)
    _fill_tile_rows(accum, zbuf, base)
    pltpu.sync_copy(dst_hbm.at[wid], dst_v)
    plsc.subcore_barrier()

    def body(j, _):
        pltpu.sync_copy(ones_v, accum.at[dst_v.at[j]], add=True)
        return 0
    lax.fori_loop(0, DCH, body, 0)
    plsc.subcore_barrier()
    pltpu.sync_copy(accum.at[pl.ds(base, RPT)],
                    out_hbm.at[cid, pl.ds(base, RPT)])


def _deg(dst_rs):
    k = pl.kernel(
        _deg_body,
        out_type=jax.ShapeDtypeStruct((NC, N_PAD, DEGW), jnp.float32),
        mesh=_MESH(),
        compiler_params=pltpu.CompilerParams(use_tc_tiling_on_sc=False),
        scratch_types=[
            pltpu.VMEM((DCH, CH), jnp.int32),
            pltpu.VMEM((CH, DEGW), jnp.float32),
            pltpu.VMEM((CH, DEGW), jnp.float32),
            pltpu.VMEM_SHARED((N_PAD, DEGW), jnp.float32),
            pltpu.SemaphoreType.DMA,
        ],
    )
    return k(dst_rs)


def _scat_body(g_hbm, src_hbm, dst_hbm, out_hbm, src_v, dst_v, rowbuf, zbuf,
               accum, gstage, *sems):
    cid = lax.axis_index("c")
    sid = lax.axis_index("s")
    base = sid * RPT
    gsems, ssems = sems[:NBUF], sems[NBUF:]
    rpt_n = N // NS

    _zero_rows(zbuf, CH, QW)
    pltpu.sync_copy(src_hbm.at[sid], src_v)
    pltpu.sync_copy(dst_hbm.at[sid], dst_v)

    for qi in range(2):
        q = cid * 2 + qi
        pltpu.sync_copy(g_hbm.at[q, pl.ds(sid * rpt_n, rpt_n)],
                        gstage.at[pl.ds(sid * rpt_n, rpt_n)])
        _fill_tile_rows(accum, zbuf, base)
        plsc.subcore_barrier()

        def body(jo, _):
            j = jo * NBUF
            gs, ss = [], []
            for b in range(NBUF):
                gs.append(pltpu.async_copy(gstage.at[src_v.at[j + b]],
                                           rowbuf.at[b], gsems[b]))
            for b in range(NBUF):
                gs[b].wait()
                ss.append(pltpu.async_copy(rowbuf.at[b],
                                           accum.at[dst_v.at[j + b]],
                                           ssems[b], add=True))
            for b in range(NBUF):
                ss[b].wait()
            return 0
        lax.fori_loop(0, SCH // NBUF, body, 0)
        plsc.subcore_barrier()
        pltpu.sync_copy(accum.at[pl.ds(base, RPT)],
                        out_hbm.at[q, pl.ds(base, RPT)])
        plsc.subcore_barrier()


def _scat(g4, src_rs, dst_rs):
    k = pl.kernel(
        _scat_body,
        out_type=jax.ShapeDtypeStruct((4, N_PAD, QW), jnp.float32),
        mesh=_MESH(),
        compiler_params=pltpu.CompilerParams(use_tc_tiling_on_sc=False),
        scratch_types=[
            pltpu.VMEM((SCH, CH), jnp.int32),
            pltpu.VMEM((SCH, CH), jnp.int32),
            pltpu.VMEM((NBUF, CH, QW), jnp.float32),
            pltpu.VMEM((CH, QW), jnp.float32),
            pltpu.VMEM_SHARED((N_PAD, QW), jnp.float32),
            pltpu.VMEM_SHARED((N, QW), jnp.float32),
        ] + [pltpu.SemaphoreType.DMA] * (2 * NBUF),
    )
    return k(g4, src_rs, dst_rs)


def _final_body(ps_hbm, pd_hbm, sl_hbm, dl_hbm, ea_hbm, out_hbm,
                ps_v, pd_v, sl_v, dl_v, ea_v, out_v, sem):
    cid = lax.axis_index("c")
    sid = lax.axis_index("s")
    wid = cid * NS + sid

    pltpu.sync_copy(ps_hbm, ps_v)
    pltpu.sync_copy(pd_hbm, pd_v)
    pltpu.sync_copy(sl_hbm.at[wid], sl_v)
    pltpu.sync_copy(dl_hbm.at[wid], dl_v)
    pltpu.sync_copy(ea_hbm.at[wid], ea_v)

    def body(j, _):
        i16 = j * 16
        s_idx = sl_v[pl.ds(i16, 16)]
        d_idx = dl_v[pl.ds(i16, 16)]
        a = plsc.load_gather(ps_v, [s_idx])
        b = plsc.load_gather(pd_v, [d_idx])
        out_v[pl.ds(i16, 16)] = a + b + ea_v[pl.ds(i16, 16)]
        return 0
    lax.fori_loop(0, LPW // 16, body, 0, unroll=4)
    pltpu.sync_copy(out_v, out_hbm.at[wid])


def _final(ps, pd, sl_rs, dl_rs, ea_rs):
    k = pl.kernel(
        _final_body,
        out_type=jax.ShapeDtypeStruct((NW, LPW), jnp.float32),
        mesh=_MESH(),
        compiler_params=pltpu.CompilerParams(needs_layout_passes=False),
        scratch_types=[
            pltpu.VMEM((N,), jnp.float32),
            pltpu.VMEM((N,), jnp.float32),
            pltpu.VMEM((LPW,), jnp.int32),
            pltpu.VMEM((LPW,), jnp.int32),
            pltpu.VMEM((LPW,), jnp.float32),
            pltpu.VMEM((LPW,), jnp.float32),
            pltpu.SemaphoreType.DMA,
        ],
    )
    return k(ps, pd, sl_rs, dl_rs, ea_rs)


def kernel(x, edge_index, edge_label_index, edge_attr, W1, b1, W2, b2,
           convW0, convb0, Wih0, Whh0, bih0, bhh0, prev0,
           convW1, convb1, Wih1, Whh1, bih1, bhh1, prev1, Wp, bp):
    f32 = jnp.float32

    src = edge_index[0]
    dst = edge_index[1]
    pad = E_PAD - E
    src_p = jnp.concatenate([src, jnp.zeros((pad,), jnp.int32)])
    dst_p = jnp.concatenate([dst, jnp.full((pad,), N, jnp.int32)])
    src_rs = src_p.reshape(NS, SCH, CH)
    dst_rs = dst_p.reshape(NS, SCH, CH)
    dst_rs32 = dst_p.reshape(NW, DCH, CH)
    sl_rs = edge_label_index[0].reshape(NW, LPW)
    dl_rs = edge_label_index[1].reshape(NW, LPW)

    b1r = b1.reshape(1, H)
    b2r = b2.reshape(1, H)
    wsd = jnp.concatenate(
        [Wp[0, :H].reshape(H, 1), Wp[0, H:2 * H].reshape(H, 1),
         jnp.zeros((H, 6), f32)], axis=1)
    wpe = jnp.concatenate([Wp[0, 2 * H:].reshape(EA, 1),
                           jnp.zeros((EA, 7), f32)], axis=1)
    bp8 = jnp.concatenate([bp.reshape(1, 1), jnp.zeros((1, 7), f32)], axis=1)

    deg2 = _deg(dst_rs32)
    h = _mlp(x, W1.T, b1r, W2.T, b2r)

    embs = []
    for cw, cb, wih, whh, bih, bhh, prev in (
            (convW0, convb0, Wih0, Whh0, bih0, bhh0, prev0),
            (convW1, convb1, Wih1, Whh1, bih1, bhh1, prev1)):
        g = _gmsg(h, cw, deg2)
        acc2 = _scat(g, src_rs, dst_rs)
        h = _gru(acc2, g, deg2, cb.reshape(1, H), prev,
                 wih.T, whh.T, bih.reshape(1, 3 * H), bhh.reshape(1, 3 * H))
        embs.append(h)

    p2 = _proj(h, wsd)
    eap = _eap(edge_attr, wpe, bp8)
    out = _final(p2[:, 0], p2[:, 1], sl_rs, dl_rs,
                 eap[:, 0].reshape(NW, LPW)).reshape(L)
    return (out, embs[0], embs[1])


# speedup vs baseline: 1.4977x; 1.0268x over previous
import functools

import jax
import jax.numpy as jnp
from jax import lax
from jax.experimental import pallas as pl
from jax.experimental.pallas import tpu as pltpu
from jax.experimental.pallas import tpu_sc as plsc

N = 10000
E = 320000
L = 320000
H = 128
EA = 16

NC = 2
NS = 16
NW = NC * NS
N_PAD = 10112
RPT = N_PAD // NS
CH = 128
NBUF = 8
E_PAD = 327680
DCH = 80
SCH = 160
HH = H // 2
QW = H // 4
LPW = L // NW
DEGW = 16

_NEG_SLOPE = 0.01


def _leaky(x):
    return jnp.where(x >= 0, x, x * _NEG_SLOPE)


_ROWS = 1000
_GRID = N // _ROWS


def _dis_from_deg(deg_ref):
    deg = deg_ref[0, :, 0:1] + deg_ref[1, :, 0:1] + 1.0
    return lax.rsqrt(deg)


def _emit_g(g_ref, hnew, cw_ref, dis):
    gm = jnp.dot(hnew, cw_ref[...],
                 preferred_element_type=jnp.float32) * dis
    for q in range(4):
        g_ref[q] = gm[:, q * QW:(q + 1) * QW]


def _mlp_g_body(x_ref, w1t_ref, b1_ref, w2t_ref, b2_ref, cw_ref, deg_ref,
                h_ref, g_ref):
    h = _leaky(jnp.dot(x_ref[...], w1t_ref[...],
                       preferred_element_type=jnp.float32) + b1_ref[...])
    h2 = _leaky(jnp.dot(h, w2t_ref[...],
                        preferred_element_type=jnp.float32) + b2_ref[...])
    h_ref[...] = h2
    _emit_g(g_ref, h2, cw_ref, _dis_from_deg(deg_ref))


def _mlp_g(x, w1t, b1, w2t, b2, cw, deg2):
    return pl.pallas_call(
        _mlp_g_body,
        grid=(_GRID,),
        in_specs=[
            pl.BlockSpec((_ROWS, H), lambda i: (i, 0)),
            pl.BlockSpec((H, H), lambda i: (0, 0)),
            pl.BlockSpec((1, H), lambda i: (0, 0)),
            pl.BlockSpec((H, H), lambda i: (0, 0)),
            pl.BlockSpec((1, H), lambda i: (0, 0)),
            pl.BlockSpec((H, H), lambda i: (0, 0)),
            pl.BlockSpec((2, _ROWS, DEGW), lambda i: (0, i, 0)),
        ],
        out_specs=[
            pl.BlockSpec((_ROWS, H), lambda i: (i, 0)),
            pl.BlockSpec((4, _ROWS, QW), lambda i: (0, i, 0)),
        ],
        out_shape=[
            jax.ShapeDtypeStruct((N, H), jnp.float32),
            jax.ShapeDtypeStruct((4, N, QW), jnp.float32),
        ],
    )(x, w1t, b1, w2t, b2, cw, deg2)


def _gru_new_h(acc_ref, g_ref, deg_ref, cb_ref, prev_ref, wiht_ref, whht_ref,
               bih_ref, bhh_ref, dis):
    full = jnp.concatenate([acc_ref[q] + g_ref[q] for q in range(4)],
                           axis=1)
    conv = dis * full + cb_ref[...]
    a = _leaky(conv)
    prev = prev_ref[...]
    gi = jnp.dot(a, wiht_ref[...], preferred_element_type=jnp.float32) + bih_ref[...]
    gh = jnp.dot(prev, whht_ref[...], preferred_element_type=jnp.float32) + bhh_ref[...]
    i_r, i_z, i_n = gi[:, :H], gi[:, H:2 * H], gi[:, 2 * H:]
    h_r, h_z, h_n = gh[:, :H], gh[:, H:2 * H], gh[:, 2 * H:]
    r = jax.nn.sigmoid(i_r + h_r)
    z = jax.nn.sigmoid(i_z + h_z)
    n = jnp.tanh(i_n + r * h_n)
    return (1.0 - z) * n + z * prev


_GRU_SPECS = [
    pl.BlockSpec((4, _ROWS, QW), lambda i: (0, i, 0)),
    pl.BlockSpec((4, _ROWS, QW), lambda i: (0, i, 0)),
    pl.BlockSpec((2, _ROWS, DEGW), lambda i: (0, i, 0)),
    pl.BlockSpec((1, H), lambda i: (0, 0)),
    pl.BlockSpec((_ROWS, H), lambda i: (i, 0)),
    pl.BlockSpec((H, 3 * H), lambda i: (0, 0)),
    pl.BlockSpec((H, 3 * H), lambda i: (0, 0)),
    pl.BlockSpec((1, 3 * H), lambda i: (0, 0)),
    pl.BlockSpec((1, 3 * H), lambda i: (0, 0)),
]


def _gru_g_body(acc_ref, g_ref, deg_ref, cb_ref, prev_ref, wiht_ref, whht_ref,
                bih_ref, bhh_ref, cw_ref, h_ref, gn_ref):
    dis = _dis_from_deg(deg_ref)
    hnew = _gru_new_h(acc_ref, g_ref, deg_ref, cb_ref, prev_ref, wiht_ref,
                      whht_ref, bih_ref, bhh_ref, dis)
    h_ref[...] = hnew
    _emit_g(gn_ref, hnew, cw_ref, dis)


def _gru_g(acc2, g, deg2, cb, prev, wiht, whht, bih, bhh, cw_next):
    return pl.pallas_call(
        _gru_g_body,
        grid=(_GRID,),
        in_specs=_GRU_SPECS + [pl.BlockSpec((H, H), lambda i: (0, 0))],
        out_specs=[
            pl.BlockSpec((_ROWS, H), lambda i: (i, 0)),
            pl.BlockSpec((4, _ROWS, QW), lambda i: (0, i, 0)),
        ],
        out_shape=[
            jax.ShapeDtypeStruct((N, H), jnp.float32),
            jax.ShapeDtypeStruct((4, N, QW), jnp.float32),
        ],
    )(acc2, g, deg2, cb, prev, wiht, whht, bih, bhh, cw_next)


def _gru_proj_body(acc_ref, g_ref, deg_ref, cb_ref, prev_ref, wiht_ref,
                   whht_ref, bih_ref, bhh_ref, wsd_ref, h_ref, p_ref):
    dis = _dis_from_deg(deg_ref)
    hnew = _gru_new_h(acc_ref, g_ref, deg_ref, cb_ref, prev_ref, wiht_ref,
                      whht_ref, bih_ref, bhh_ref, dis)
    h_ref[...] = hnew
    p_ref[...] = jnp.dot(hnew, wsd_ref[...],
                         preferred_element_type=jnp.float32)


def _gru_proj(acc2, g, deg2, cb, prev, wiht, whht, bih, bhh, wsd):
    return pl.pallas_call(
        _gru_proj_body,
        grid=(_GRID,),
        in_specs=_GRU_SPECS + [pl.BlockSpec((H, 8), lambda i: (0, 0))],
        out_specs=[
            pl.BlockSpec((_ROWS, H), lambda i: (i, 0)),
            pl.BlockSpec((_ROWS, 8), lambda i: (i, 0)),
        ],
        out_shape=[
            jax.ShapeDtypeStruct((N, H), jnp.float32),
            jax.ShapeDtypeStruct((N, 8), jnp.float32),
        ],
    )(acc2, g, deg2, cb, prev, wiht, whht, bih, bhh, wsd)


_EROWS = 8000


def _eap_body(ea_ref, w_ref, bp_ref, o_ref):
    o_ref[...] = jnp.dot(ea_ref[...], w_ref[...],
                         preferred_element_type=jnp.float32) + bp_ref[...]


def _eap(edge_attr, wpe, bp):
    return pl.pallas_call(
        _eap_body,
        grid=(L // _EROWS,),
        in_specs=[
            pl.BlockSpec((_EROWS, EA), lambda i: (i, 0)),
            pl.BlockSpec((EA, 8), lambda i: (0, 0)),
            pl.BlockSpec((1, 8), lambda i: (0, 0)),
        ],
        out_specs=pl.BlockSpec((_EROWS, 8), lambda i: (i, 0)),
        out_shape=jax.ShapeDtypeStruct((L, 8), jnp.float32),
    )(edge_attr, wpe, bp)


_MESH = functools.partial(plsc.VectorSubcoreMesh,
                          core_axis_name="c", subcore_axis_name="s",
                          num_cores=NC, num_subcores=NS)


def _zero_rows(buf, nrows, width):
    def body(i, _):
        for k in range(width // 16):
            buf[i, pl.ds(k * 16, 16)] = jnp.zeros((16,), jnp.float32)
        return 0
    lax.fori_loop(0, nrows, body, 0, unroll=2)


def _fill_tile_rows(accum, src2d, base):
    for off in (0, 128, 256, 384):
        pltpu.sync_copy(src2d, accum.at[pl.ds(base + off, 128)])
    pltpu.sync_copy(src2d.at[pl.ds(0, 120)], accum.at[pl.ds(base + 512, 120)])


def _deg_body(dst_hbm, out_hbm, dst_v, ones_v, zbuf, accum, sem):
    cid = lax.axis_index("c")
    sid = lax.axis_index("s")
    wid = cid * NS + sid
    base = sid * RPT

    _zero_rows(zbuf, CH, DEGW)
    def ones_body(i, _):
        ones_v[i, pl.ds(0, DEGW)] = jnp.ones((DEGW,), jnp.float32)
        return 0
    lax.fori_loop(0, CH, ones_body, 0, unroll=2)
    _fill_tile_rows(accum, zbuf, base)
    pltpu.sync_copy(dst_hbm.at[wid], dst_v)
    plsc.subcore_barrier()

    def body(j, _):
        pltpu.sync_copy(ones_v, accum.at[dst_v.at[j]], add=True)
        return 0
    lax.fori_loop(0, DCH, body, 0)
    plsc.subcore_barrier()
    pltpu.sync_copy(accum.at[pl.ds(base, RPT)],
                    out_hbm.at[cid, pl.ds(base, RPT)])


def _deg(dst_rs):
    k = pl.kernel(
        _deg_body,
        out_type=jax.ShapeDtypeStruct((NC, N_PAD, DEGW), jnp.float32),
        mesh=_MESH(),
        compiler_params=pltpu.CompilerParams(use_tc_tiling_on_sc=False),
        scratch_types=[
            pltpu.VMEM((DCH, CH), jnp.int32),
            pltpu.VMEM((CH, DEGW), jnp.float32),
            pltpu.VMEM((CH, DEGW), jnp.float32),
            pltpu.VMEM_SHARED((N_PAD, DEGW), jnp.float32),
            pltpu.SemaphoreType.DMA,
        ],
    )
    return k(dst_rs)


def _scat_body(g_hbm, src_hbm, dst_hbm, out_hbm, src_v, dst_v, rowbuf, zbuf,
               accum, gstage, *sems):
    cid = lax.axis_index("c")
    sid = lax.axis_index("s")
    base = sid * RPT
    gsems, ssems = sems[:NBUF], sems[NBUF:]
    rpt_n = N // NS

    _zero_rows(zbuf, CH, QW)
    pltpu.sync_copy(src_hbm.at[sid], src_v)
    pltpu.sync_copy(dst_hbm.at[sid], dst_v)

    for qi in range(2):
        q = cid * 2 + qi
        pltpu.sync_copy(g_hbm.at[q, pl.ds(sid * rpt_n, rpt_n)],
                        gstage.at[pl.ds(sid * rpt_n, rpt_n)])
        _fill_tile_rows(accum, zbuf, base)
        plsc.subcore_barrier()

        def body(jo, _):
            j = jo * NBUF
            gs, ss = [], []
            for b in range(NBUF):
                gs.append(pltpu.async_copy(gstage.at[src_v.at[j + b]],
                                           rowbuf.at[b], gsems[b]))
            for b in range(NBUF):
                gs[b].wait()
                ss.append(pltpu.async_copy(rowbuf.at[b],
                                           accum.at[dst_v.at[j + b]],
                                           ssems[b], add=True))
            for b in range(NBUF):
                ss[b].wait()
            return 0
        lax.fori_loop(0, SCH // NBUF, body, 0)
        plsc.subcore_barrier()
        pltpu.sync_copy(accum.at[pl.ds(base, RPT)],
                        out_hbm.at[q, pl.ds(base, RPT)])
        plsc.subcore_barrier()


def _scat(g4, src_rs, dst_rs):
    k = pl.kernel(
        _scat_body,
        out_type=jax.ShapeDtypeStruct((4, N_PAD, QW), jnp.float32),
        mesh=_MESH(),
        compiler_params=pltpu.CompilerParams(use_tc_tiling_on_sc=False),
        scratch_types=[
            pltpu.VMEM((SCH, CH), jnp.int32),
            pltpu.VMEM((SCH, CH), jnp.int32),
            pltpu.VMEM((NBUF, CH, QW), jnp.float32),
            pltpu.VMEM((CH, QW), jnp.float32),
            pltpu.VMEM_SHARED((N_PAD, QW), jnp.float32),
            pltpu.VMEM_SHARED((N, QW), jnp.float32),
        ] + [pltpu.SemaphoreType.DMA] * (2 * NBUF),
    )
    return k(g4, src_rs, dst_rs)


def _final_body(ps_hbm, pd_hbm, sl_hbm, dl_hbm, ea_hbm, out_hbm,
                ps_v, pd_v, sl_v, dl_v, ea_v, out_v, sem):
    cid = lax.axis_index("c")
    sid = lax.axis_index("s")
    wid = cid * NS + sid

    pltpu.sync_copy(ps_hbm, ps_v)
    pltpu.sync_copy(pd_hbm, pd_v)
    pltpu.sync_copy(sl_hbm.at[wid], sl_v)
    pltpu.sync_copy(dl_hbm.at[wid], dl_v)
    pltpu.sync_copy(ea_hbm.at[wid], ea_v)

    def body(j, _):
        i16 = j * 16
        s_idx = sl_v[pl.ds(i16, 16)]
        d_idx = dl_v[pl.ds(i16, 16)]
        a = plsc.load_gather(ps_v, [s_idx])
        b = plsc.load_gather(pd_v, [d_idx])
        out_v[pl.ds(i16, 16)] = a + b + ea_v[pl.ds(i16, 16)]
        return 0
    lax.fori_loop(0, LPW // 16, body, 0, unroll=4)
    pltpu.sync_copy(out_v, out_hbm.at[wid])


def _final(ps, pd, sl_rs, dl_rs, ea_rs):
    k = pl.kernel(
        _final_body,
        out_type=jax.ShapeDtypeStruct((NW, LPW), jnp.float32),
        mesh=_MESH(),
        compiler_params=pltpu.CompilerParams(needs_layout_passes=False),
        scratch_types=[
            pltpu.VMEM((N,), jnp.float32),
            pltpu.VMEM((N,), jnp.float32),
            pltpu.VMEM((LPW,), jnp.int32),
            pltpu.VMEM((LPW,), jnp.int32),
            pltpu.VMEM((LPW,), jnp.float32),
            pltpu.VMEM((LPW,), jnp.float32),
            pltpu.SemaphoreType.DMA,
        ],
    )
    return k(ps, pd, sl_rs, dl_rs, ea_rs)


def kernel(x, edge_index, edge_label_index, edge_attr, W1, b1, W2, b2,
           convW0, convb0, Wih0, Whh0, bih0, bhh0, prev0,
           convW1, convb1, Wih1, Whh1, bih1, bhh1, prev1, Wp, bp):
    f32 = jnp.float32

    src = edge_index[0]
    dst = edge_index[1]
    pad = E_PAD - E
    src_p = jnp.concatenate([src, jnp.zeros((pad,), jnp.int32)])
    dst_p = jnp.concatenate([dst, jnp.full((pad,), N, jnp.int32)])
    src_rs = src_p.reshape(NS, SCH, CH)
    dst_rs = dst_p.reshape(NS, SCH, CH)
    dst_rs32 = dst_p.reshape(NW, DCH, CH)
    sl_rs = edge_label_index[0].reshape(NW, LPW)
    dl_rs = edge_label_index[1].reshape(NW, LPW)

    b1r = b1.reshape(1, H)
    b2r = b2.reshape(1, H)
    wsd = jnp.concatenate(
        [Wp[0, :H].reshape(H, 1), Wp[0, H:2 * H].reshape(H, 1),
         jnp.zeros((H, 6), f32)], axis=1)
    wpe = jnp.concatenate([Wp[0, 2 * H:].reshape(EA, 1),
                           jnp.zeros((EA, 7), f32)], axis=1)
    bp8 = jnp.concatenate([bp.reshape(1, 1), jnp.zeros((1, 7), f32)], axis=1)

    deg2 = _deg(dst_rs32)
    _, g0 = _mlp_g(x, W1.T, b1r, W2.T, b2r, convW0, deg2)
    acc0 = _scat(g0, src_rs, dst_rs)
    h1, g1 = _gru_g(acc0, g0, deg2, convb0.reshape(1, H), prev0,
                    Wih0.T, Whh0.T, bih0.reshape(1, 3 * H),
                    bhh0.reshape(1, 3 * H), convW1)
    acc1 = _scat(g1, src_rs, dst_rs)
    h2, p2 = _gru_proj(acc1, g1, deg2, convb1.reshape(1, H), prev1,
                       Wih1.T, Whh1.T, bih1.reshape(1, 3 * H),
                       bhh1.reshape(1, 3 * H), wsd)
    eap = _eap(edge_attr, wpe, bp8)
    out = _final(p2[:, 0], p2[:, 1], sl_rs, dl_rs,
                 eap[:, 0].reshape(NW, LPW)).reshape(L)
    return (out, h1, h2)
